# baseline (speedup 1.0000x reference)
import functools

import jax
import jax.numpy as jnp
from jax import lax
from jax.experimental import pallas as pl
from jax.experimental.pallas import tpu as pltpu
from jax.experimental.pallas import tpu_sc as plsc

NC = 2
NS = 16
NW = NC * NS
CHUNK = 128


def _sc_mesh():
    return plsc.VectorSubcoreMesh(core_axis_name="c", subcore_axis_name="s")


def _chunk_split(E):
    nch = E // CHUNK
    bc = nch // NW
    extra = nch - bc * NW
    return nch, bc, extra


def _make_gather(E, N, H):
    nch, bc, extra = _chunk_split(E)
    HP = H // 2
    sds = jax.ShapeDtypeStruct((E, HP), jnp.int32)
    NB = 4

    @functools.partial(
        pl.kernel,
        mesh=_sc_mesh(),
        out_type=(sds, sds),
        compiler_params=pltpu.CompilerParams(use_tc_tiling_on_sc=False),
        scratch_types=[
            pltpu.VMEM(((bc + 1) * CHUNK,), jnp.int32),
            pltpu.VMEM(((bc + 1) * CHUNK,), jnp.int32),
            pltpu.VMEM((NB * CHUNK, HP), jnp.int32),
            pltpu.VMEM((NB * CHUNK, HP), jnp.int32),
            pltpu.SemaphoreType.DMA((NB,)),
            pltpu.SemaphoreType.DMA((NB,)),
            pltpu.SemaphoreType.DMA((NB,)),
            pltpu.SemaphoreType.DMA((NB,)),
        ],
    )
    def gather_kernel(ta, tb, row, col, outa, outb, idxr, idxc, bufa, bufb,
                      sga, sgb, swa, swb):
        wid = lax.axis_index("s") * NC + lax.axis_index("c")
        cs = wid * bc + jnp.minimum(wid, extra)
        cnt = bc + jnp.where(wid < extra, 1, 0)

        pltpu.sync_copy(row.at[pl.ds(cs * CHUNK, bc * CHUNK)],
                        idxr.at[pl.ds(0, bc * CHUNK)])
        pltpu.sync_copy(col.at[pl.ds(cs * CHUNK, bc * CHUNK)],
                        idxc.at[pl.ds(0, bc * CHUNK)])
        if extra:
            @pl.when(wid < extra)
            def _():
                off = (cs + bc) * CHUNK
                pltpu.sync_copy(row.at[pl.ds(off, CHUNK)],
                                idxr.at[pl.ds(bc * CHUNK, CHUNK)])
                pltpu.sync_copy(col.at[pl.ds(off, CHUNK)],
                                idxc.at[pl.ds(bc * CHUNK, CHUNK)])

        def slot(t):
            return (t % NB) * CHUNK

        def gather_pair(t):
            s = t % NB
            return (
                pltpu.make_async_copy(ta.at[idxr.at[pl.ds(t * CHUNK, CHUNK)]],
                                      bufa.at[pl.ds(slot(t), CHUNK)], sga.at[s]),
                pltpu.make_async_copy(tb.at[idxc.at[pl.ds(t * CHUNK, CHUNK)]],
                                      bufb.at[pl.ds(slot(t), CHUNK)], sgb.at[s]),
            )

        def write_pair(t):
            off = (cs + t) * CHUNK
            s = t % NB
            return (
                pltpu.make_async_copy(bufa.at[pl.ds(slot(t), CHUNK)],
                                      outa.at[pl.ds(off, CHUNK)], swa.at[s]),
                pltpu.make_async_copy(bufb.at[pl.ds(slot(t), CHUNK)],
                                      outb.at[pl.ds(off, CHUNK)], swb.at[s]),
            )

        def start(ops):
            for op in ops:
                op.start()

        def wait(ops):
            for op in ops:
                op.wait()

        @pl.loop(0, cnt)
        def _(t):
            @pl.when(t >= NB)
            def _():
                wait(write_pair(t - NB))

            start(gather_pair(t))

            @pl.when(t >= 1)
            def _():
                wait(gather_pair(t - 1))
                start(write_pair(t - 1))

        wait(gather_pair(cnt - 1))
        start(write_pair(cnt - 1))

        @pl.loop(jnp.maximum(cnt - NB, 0), cnt)
        def _(t):
            wait(write_pair(t))

    return gather_kernel


def _make_scatter(E, N, H):
    nch, bc, extra = _chunk_split(E)
    rps = (N // NS) & ~7
    rrem = N - NS * rps

    @functools.partial(
        pl.kernel,
        mesh=_sc_mesh(),
        out_type=jax.ShapeDtypeStruct((NC, N, H), jnp.float32),
        scratch_types=[
            pltpu.VMEM((bc + 1, CHUNK), jnp.int32),
            pltpu.VMEM((2 * CHUNK, H), jnp.float32),
            pltpu.VMEM_SHARED((N, H), jnp.float32),
            pltpu.SemaphoreType.DMA,
            pltpu.SemaphoreType.DMA((2,)),
            pltpu.SemaphoreType.DMA((2,)),
        ],
    )
    def scatter_kernel(e_h, row_h, zeros_h, out_h, idx2d, ebuf, agg,
                       semi, seml, sems):
        cid = lax.axis_index("c")
        sid = lax.axis_index("s")
        wid = sid * NC + cid
        rbase = sid * rps
        cs = wid * bc + jnp.minimum(wid, extra)
        cnt = bc + jnp.where(wid < extra, 1, 0)

        @pl.loop(0, cnt)
        def _(t):
            pltpu.async_copy(row_h.at[pl.ds((cs + t) * CHUNK, CHUNK)],
                             idx2d.at[t], semi)

        pltpu.sync_copy(zeros_h.at[pl.ds(rbase, rps)], agg.at[pl.ds(rbase, rps)])
        if rrem:
            @pl.when(sid == 0)
            def _():
                pltpu.sync_copy(zeros_h.at[pl.ds(NS * rps, rrem)],
                                agg.at[pl.ds(NS * rps, rrem)])

        @pl.loop(0, cnt)
        def _(t):
            pltpu.make_async_copy(row_h.at[pl.ds((cs + t) * CHUNK, CHUNK)],
                                  idx2d.at[t], semi).wait()

        plsc.subcore_barrier()

        NB = 2

        def slot(t):
            return (t % NB) * CHUNK

        def load_op(t):
            return pltpu.make_async_copy(e_h.at[pl.ds((cs + t) * CHUNK, CHUNK)],
                                         ebuf.at[pl.ds(slot(t), CHUNK)],
                                         seml.at[t % NB])

        def scat_op(t):
            return pltpu.make_async_copy(ebuf.at[pl.ds(slot(t), CHUNK)],
                                         agg.at[idx2d.at[t]], sems.at[t % NB])

        @pl.loop(0, cnt)
        def _(t):
            @pl.when(t >= NB)
            def _():
                scat_op(t - NB).wait()

            load_op(t).start()

            @pl.when(t >= 1)
            def _():
                load_op(t - 1).wait()
                scat_op(t - 1).start(add=True)

        load_op(cnt - 1).wait()
        scat_op(cnt - 1).start(add=True)

        @pl.loop(jnp.maximum(cnt - NB, 0), cnt)
        def _(t):
            scat_op(t).wait()

        plsc.subcore_barrier()
        pltpu.sync_copy(agg.at[pl.ds(rbase, rps)], out_h.at[cid, pl.ds(rbase, rps)])
        if rrem:
            @pl.when(sid == 0)
            def _():
                pltpu.sync_copy(agg.at[pl.ds(NS * rps, rrem)],
                                out_h.at[cid, pl.ds(NS * rps, rrem)])

    return scatter_kernel


def _unpack_pair(w):
    lo = jax.lax.bitcast_convert_type(jnp.left_shift(w, 16), jnp.float32)
    hi = jax.lax.bitcast_convert_type(
        jnp.bitwise_and(w, jnp.int32(-65536)), jnp.float32)
    return lo, hi


def _pack_pair(lo, hi):
    def rn(x):
        b = jax.lax.bitcast_convert_type(x, jnp.int32)
        return jax.lax.shift_right_logical(
            b + 0x7FFF + jnp.bitwise_and(jax.lax.shift_right_logical(b, 16), 1),
            16)
    return jnp.bitwise_or(rn(lo), jnp.left_shift(rn(hi), 16))


def _edge_body(ga_ref, gb_ref, e_ref, w1c_ref, w2_ref, b2_ref, out_ref):
    ev = e_ref[...]
    ga_lo, ga_hi = _unpack_pair(ga_ref[...])
    gb_lo, gb_hi = _unpack_pair(gb_ref[...])
    g = jnp.concatenate([ga_lo + gb_lo, ga_hi + gb_hi], axis=1)
    x = g + jnp.dot(ev.astype(jnp.bfloat16), w1c_ref[...],
                    preferred_element_type=jnp.float32)
    x = jnp.maximum(x, 0.0)
    out_ref[...] = jnp.dot(
        x.astype(jnp.bfloat16), w2_ref[...],
        preferred_element_type=jnp.float32) + (ev + b2_ref[...])


def _proj_body(h_ref, w1a_ref, w1b_ref, b1_ref, oa_ref, ob_ref):
    hv = h_ref[...]
    HP = hv.shape[1] // 2
    oa = jnp.dot(
        hv, w1a_ref[...], preferred_element_type=jnp.float32) + b1_ref[...]
    ob = jnp.dot(hv, w1b_ref[...], preferred_element_type=jnp.float32)
    oa_ref[...] = _pack_pair(oa[:, :HP], oa[:, HP:])
    ob_ref[...] = _pack_pair(ob[:, :HP], ob[:, HP:])


def _node_body(h_ref, a0_ref, a1_ref, w1a_ref, w1b_ref, b1_ref, w2_ref, b2_ref,
               out_ref):
    agg = a0_ref[...] + a1_ref[...]
    x = (jnp.dot(h_ref[...], w1a_ref[...], preferred_element_type=jnp.float32)
         + jnp.dot(agg, w1b_ref[...], preferred_element_type=jnp.float32)
         + b1_ref[...])
    x = jnp.maximum(x, 0.0)
    out_ref[...] = jnp.dot(
        x, w2_ref[...], preferred_element_type=jnp.float32) + b2_ref[...] + h_ref[...]


def kernel(h, e, edge_index, edge_w1, edge_b1, edge_w2, edge_b2,
           gate_w1, gate_b1, gate_w2, gate_b2,
           node_w1, node_b1, node_w2, node_b2):
    N, H = h.shape
    E = e.shape[0]
    assert E % CHUNK == 0 and N % 8 == 0

    row = edge_index[0]
    col = edge_index[1]
    w1a, w1b, w1c = edge_w1[:H], edge_w1[H:2 * H], edge_w1[2 * H:]
    b1 = edge_b1.reshape(1, H)
    b2 = edge_b2.reshape(1, H)

    PT = 1000
    proj = pl.pallas_call(
        _proj_body,
        grid=(N // PT,),
        in_specs=[
            pl.BlockSpec((PT, H), lambda i: (i, 0)),
            pl.BlockSpec((H, H), lambda i: (0, 0)),
            pl.BlockSpec((H, H), lambda i: (0, 0)),
            pl.BlockSpec((1, H), lambda i: (0, 0)),
        ],
        out_specs=[
            pl.BlockSpec((PT, H // 2), lambda i: (i, 0)),
            pl.BlockSpec((PT, H // 2), lambda i: (i, 0)),
        ],
        out_shape=[
            jax.ShapeDtypeStruct((N, H // 2), jnp.int32),
            jax.ShapeDtypeStruct((N, H // 2), jnp.int32),
        ],
    )
    ha, hb = proj(h, w1a, w1b, b1)

    ga, gb = _make_gather(E, N, H)(ha, hb, row, col)

    aggp = _make_scatter(E, N, H)(e, row, jnp.zeros((N, H), jnp.float32))

    ET = 2000
    edge_feat = pl.pallas_call(
        _edge_body,
        grid=(E // ET,),
        in_specs=[
            pl.BlockSpec((ET, H // 2), lambda i: (i, 0)),
            pl.BlockSpec((ET, H // 2), lambda i: (i, 0)),
            pl.BlockSpec((ET, H), lambda i: (i, 0)),
            pl.BlockSpec((H, H), lambda i: (0, 0)),
            pl.BlockSpec((H, H), lambda i: (0, 0)),
            pl.BlockSpec((1, H), lambda i: (0, 0)),
        ],
        out_specs=pl.BlockSpec((ET, H), lambda i: (i, 0)),
        out_shape=jax.ShapeDtypeStruct((E, H), jnp.float32),
    )(ga, gb, e, w1c.astype(jnp.bfloat16), edge_w2.astype(jnp.bfloat16), b2)

    nw1a, nw1b = node_w1[:H], node_w1[H:]
    NT = 1000
    h_out = pl.pallas_call(
        _node_body,
        grid=(N // NT,),
        in_specs=[
            pl.BlockSpec((NT, H), lambda i: (i, 0)),
            pl.BlockSpec((NT, H), lambda i: (i, 0)),
            pl.BlockSpec((NT, H), lambda i: (i, 0)),
            pl.BlockSpec((H, H), lambda i: (0, 0)),
            pl.BlockSpec((H, H), lambda i: (0, 0)),
            pl.BlockSpec((1, H), lambda i: (0, 0)),
            pl.BlockSpec((H, H), lambda i: (0, 0)),
            pl.BlockSpec((1, H), lambda i: (0, 0)),
        ],
        out_specs=pl.BlockSpec((NT, H), lambda i: (i, 0)),
        out_shape=jax.ShapeDtypeStruct((N, H), jnp.float32),
    )(h, aggp[0], aggp[1], nw1a, nw1b, node_b1.reshape(1, H), node_w2,
      node_b2.reshape(1, H))

    return (h_out, edge_feat)


# speedup vs baseline: 1.1029x; 1.1029x over previous
import functools

import jax
import jax.numpy as jnp
from jax import lax
from jax.experimental import pallas as pl
from jax.experimental.pallas import tpu as pltpu
from jax.experimental.pallas import tpu_sc as plsc

NC = 2
NS = 16
NW = NC * NS
CHUNK = 128


def _sc_mesh():
    return plsc.VectorSubcoreMesh(core_axis_name="c", subcore_axis_name="s")


def _make_sc_fused(E, N, H):
    nch = E // CHUNK
    bc = nch // NW
    gextra = nch - bc * NW
    bs = nch // NS
    sextra = nch - bs * NS
    HP = H // 2
    rps = (N // NS) & ~7
    rrem = N - NS * rps

    NBG = 2
    NBS = 2
    NIX = 4

    out_ga = jax.ShapeDtypeStruct((E, H), jnp.float32)
    out_gb = jax.ShapeDtypeStruct((E, H), jnp.float32)
    out_agg = jax.ShapeDtypeStruct((NC, N, HP), jnp.float32)

    @functools.partial(
        pl.kernel,
        mesh=_sc_mesh(),
        out_type=(out_ga, out_gb, out_agg),
        compiler_params=pltpu.CompilerParams(use_tc_tiling_on_sc=False),
        scratch_types=[
            pltpu.VMEM((NIX, CHUNK), jnp.int32),
            pltpu.VMEM((NIX, CHUNK), jnp.int32),
            pltpu.VMEM((NIX, CHUNK), jnp.int32),
            pltpu.VMEM((NBG * CHUNK, H), jnp.float32),
            pltpu.VMEM((NBG * CHUNK, H), jnp.float32),
            pltpu.VMEM((NBS * CHUNK, HP), jnp.float32),
            pltpu.VMEM_SHARED((N, HP), jnp.float32),
            pltpu.SemaphoreType.DMA((NIX,)),
            pltpu.SemaphoreType.DMA((NIX,)),
            pltpu.SemaphoreType.DMA((NIX,)),
            pltpu.SemaphoreType.DMA((NBG,)),
            pltpu.SemaphoreType.DMA((NBG,)),
            pltpu.SemaphoreType.DMA((NBG,)),
            pltpu.SemaphoreType.DMA((NBG,)),
            pltpu.SemaphoreType.DMA((NBS,)),
            pltpu.SemaphoreType.DMA((NBS,)),
        ],
    )
    def fused_kernel(ta, tb, row, col, e_h, zeros_h,
                     outa, outb, agg_out,
                     idxgr, idxgc, idxs, bufa, bufb, ebuf, agg,
                     sixr, sixc, sixs, sga, sgb, swa, swb, sel, ssc):
        cid = lax.axis_index("c")
        sid = lax.axis_index("s")
        wid = sid * NC + cid
        gs = wid * bc + jnp.minimum(wid, gextra)
        gcnt = bc + jnp.where(wid < gextra, 1, 0)
        ss = sid * bs + jnp.minimum(sid, sextra)
        scnt = bs + jnp.where(sid < sextra, 1, 0)
        fb = cid * HP
        rbase = sid * rps

        def gidx_ops(u):
            off = (gs + u) * CHUNK
            s = u % NIX
            return (
                pltpu.make_async_copy(row.at[pl.ds(off, CHUNK)],
                                      idxgr.at[s], sixr.at[s]),
                pltpu.make_async_copy(col.at[pl.ds(off, CHUNK)],
                                      idxgc.at[s], sixc.at[s]),
            )

        def gather_ops(u):
            s = u % NBG
            return (
                pltpu.make_async_copy(ta.at[idxgr.at[u % NIX]],
                                      bufa.at[pl.ds(s * CHUNK, CHUNK)],
                                      sga.at[s]),
                pltpu.make_async_copy(tb.at[idxgc.at[u % NIX]],
                                      bufb.at[pl.ds(s * CHUNK, CHUNK)],
                                      sgb.at[s]),
            )

        def write_ops(u):
            off = (gs + u) * CHUNK
            s = u % NBG
            return (
                pltpu.make_async_copy(bufa.at[pl.ds(s * CHUNK, CHUNK)],
                                      outa.at[pl.ds(off, CHUNK)], swa.at[s]),
                pltpu.make_async_copy(bufb.at[pl.ds(s * CHUNK, CHUNK)],
                                      outb.at[pl.ds(off, CHUNK)], swb.at[s]),
            )

        def sidx_op(u):
            off = (ss + u) * CHUNK
            s = u % NIX
            return pltpu.make_async_copy(row.at[pl.ds(off, CHUNK)],
                                         idxs.at[s], sixs.at[s])

        def eload_op(u):
            off = (ss + u) * CHUNK
            s = u % NBS
            return pltpu.make_async_copy(
                e_h.at[pl.ds(off, CHUNK), pl.ds(fb, HP)],
                ebuf.at[pl.ds(s * CHUNK, CHUNK)], sel.at[s])

        def scat_op(u):
            s = u % NBS
            return pltpu.make_async_copy(ebuf.at[pl.ds(s * CHUNK, CHUNK)],
                                         agg.at[idxs.at[u % NIX]], ssc.at[s])

        def start(ops):
            for op in ops:
                op.start()

        def wait(ops):
            for op in ops:
                op.wait()

        start(gidx_ops(0))
        sidx_op(0).start()
        pltpu.sync_copy(zeros_h.at[pl.ds(rbase, rps)], agg.at[pl.ds(rbase, rps)])
        if rrem:
            @pl.when(sid == 0)
            def _():
                pltpu.sync_copy(zeros_h.at[pl.ds(NS * rps, rrem)],
                                agg.at[pl.ds(NS * rps, rrem)])
        plsc.subcore_barrier()

        tmax = jnp.maximum(gcnt, scnt)

        @pl.loop(0, tmax)
        def _(t):
            @pl.when(t < gcnt)
            def _():
                @pl.when(t + 1 < gcnt)
                def _():
                    start(gidx_ops(t + 1))

                @pl.when(t >= NBG)
                def _():
                    wait(write_ops(t - NBG))

                wait(gidx_ops(t))
                start(gather_ops(t))

                @pl.when(t >= 1)
                def _():
                    wait(gather_ops(t - 1))
                    start(write_ops(t - 1))

            @pl.when(t < scnt)
            def _():
                @pl.when(t + 1 < scnt)
                def _():
                    sidx_op(t + 1).start()

                @pl.when(t >= NBS)
                def _():
                    scat_op(t - NBS).wait()

                eload_op(t).start()

                @pl.when(t >= 1)
                def _():
                    eload_op(t - 1).wait()
                    sidx_op(t - 1).wait()
                    scat_op(t - 1).start(add=True)

        wait(gather_ops(gcnt - 1))
        start(write_ops(gcnt - 1))

        @pl.loop(jnp.maximum(gcnt - NBG, 0), gcnt)
        def _(t):
            wait(write_ops(t))

        eload_op(scnt - 1).wait()
        sidx_op(scnt - 1).wait()
        scat_op(scnt - 1).start(add=True)

        @pl.loop(jnp.maximum(scnt - NBS, 0), scnt)
        def _(t):
            scat_op(t).wait()

        plsc.subcore_barrier()
        pltpu.sync_copy(agg.at[pl.ds(rbase, rps)],
                        agg_out.at[cid, pl.ds(rbase, rps)])
        if rrem:
            @pl.when(sid == 0)
            def _():
                pltpu.sync_copy(agg.at[pl.ds(NS * rps, rrem)],
                                agg_out.at[cid, pl.ds(NS * rps, rrem)])

    return fused_kernel


def _edge_body(ga_ref, gb_ref, e_ref, w1c_ref, w2_ref, b2_ref, out_ref):
    ev = e_ref[...]
    x = (ga_ref[...] + gb_ref[...]
         + jnp.dot(ev.astype(jnp.bfloat16), w1c_ref[...],
                   preferred_element_type=jnp.float32))
    x = jnp.maximum(x, 0.0)
    out_ref[...] = jnp.dot(
        x.astype(jnp.bfloat16), w2_ref[...],
        preferred_element_type=jnp.float32) + (ev + b2_ref[...])


def _proj_body(h_ref, w1a_ref, w1b_ref, b1_ref, oa_ref, ob_ref):
    hv = h_ref[...]
    oa_ref[...] = jnp.dot(
        hv, w1a_ref[...], preferred_element_type=jnp.float32) + b1_ref[...]
    ob_ref[...] = jnp.dot(hv, w1b_ref[...], preferred_element_type=jnp.float32)


def _node_body(h_ref, a0_ref, a1_ref, w1a_ref, w1b_ref, b1_ref, w2_ref, b2_ref,
               out_ref):
    agg = jnp.concatenate([a0_ref[...], a1_ref[...]], axis=1)
    x = (jnp.dot(h_ref[...], w1a_ref[...], preferred_element_type=jnp.float32)
         + jnp.dot(agg, w1b_ref[...], preferred_element_type=jnp.float32)
         + b1_ref[...])
    x = jnp.maximum(x, 0.0)
    out_ref[...] = jnp.dot(
        x, w2_ref[...], preferred_element_type=jnp.float32) + b2_ref[...] + h_ref[...]


def kernel(h, e, edge_index, edge_w1, edge_b1, edge_w2, edge_b2,
           gate_w1, gate_b1, gate_w2, gate_b2,
           node_w1, node_b1, node_w2, node_b2):
    N, H = h.shape
    E = e.shape[0]
    assert E % CHUNK == 0 and N % 8 == 0

    row = edge_index[0]
    col = edge_index[1]
    w1a, w1b, w1c = edge_w1[:H], edge_w1[H:2 * H], edge_w1[2 * H:]
    b1 = edge_b1.reshape(1, H)
    b2 = edge_b2.reshape(1, H)

    PT = 1000
    proj = pl.pallas_call(
        _proj_body,
        grid=(N // PT,),
        in_specs=[
            pl.BlockSpec((PT, H), lambda i: (i, 0)),
            pl.BlockSpec((H, H), lambda i: (0, 0)),
            pl.BlockSpec((H, H), lambda i: (0, 0)),
            pl.BlockSpec((1, H), lambda i: (0, 0)),
        ],
        out_specs=[
            pl.BlockSpec((PT, H), lambda i: (i, 0)),
            pl.BlockSpec((PT, H), lambda i: (i, 0)),
        ],
        out_shape=[
            jax.ShapeDtypeStruct((N, H), jnp.float32),
            jax.ShapeDtypeStruct((N, H), jnp.float32),
        ],
    )
    ha, hb = proj(h, w1a, w1b, b1)

    ga, gb, aggp = _make_sc_fused(E, N, H)(
        ha, hb, row, col, e, jnp.zeros((N, H // 2), jnp.float32))

    ET = 2000
    edge_feat = pl.pallas_call(
        _edge_body,
        grid=(E // ET,),
        in_specs=[
            pl.BlockSpec((ET, H), lambda i: (i, 0)),
            pl.BlockSpec((ET, H), lambda i: (i, 0)),
            pl.BlockSpec((ET, H), lambda i: (i, 0)),
            pl.BlockSpec((H, H), lambda i: (0, 0)),
            pl.BlockSpec((H, H), lambda i: (0, 0)),
            pl.BlockSpec((1, H), lambda i: (0, 0)),
        ],
        out_specs=pl.BlockSpec((ET, H), lambda i: (i, 0)),
        out_shape=jax.ShapeDtypeStruct((E, H), jnp.float32),
    )(ga, gb, e, w1c.astype(jnp.bfloat16), edge_w2.astype(jnp.bfloat16), b2)

    nw1a, nw1b = node_w1[:H], node_w1[H:]
    NT = 1000
    h_out = pl.pallas_call(
        _node_body,
        grid=(N // NT,),
        in_specs=[
            pl.BlockSpec((NT, H), lambda i: (i, 0)),
            pl.BlockSpec((NT, H // 2), lambda i: (i, 0)),
            pl.BlockSpec((NT, H // 2), lambda i: (i, 0)),
            pl.BlockSpec((H, H), lambda i: (0, 0)),
            pl.BlockSpec((H, H), lambda i: (0, 0)),
            pl.BlockSpec((1, H), lambda i: (0, 0)),
            pl.BlockSpec((H, H), lambda i: (0, 0)),
            pl.BlockSpec((1, H), lambda i: (0, 0)),
        ],
        out_specs=pl.BlockSpec((NT, H), lambda i: (i, 0)),
        out_shape=jax.ShapeDtypeStruct((N, H), jnp.float32),
    )(h, aggp[0], aggp[1], nw1a, nw1b, node_b1.reshape(1, H), node_w2,
      node_b2.reshape(1, H))

    return (h_out, edge_feat)
